# Initial kernel scaffold; baseline (speedup 1.0000x reference)
import jax
import jax.numpy as jnp
from jax.experimental import pallas as pl


def kernel(tokens, embed_table, pos_table):
    raise NotImplementedError("write your pallas kernel here")


# speedup vs baseline: 1.1444x; 1.1444x over previous
import functools

import jax
import jax.numpy as jnp
from jax import lax
from jax.experimental import pallas as pl
from jax.experimental.pallas import tpu as pltpu
from jax.experimental.pallas import tpu_sc as plsc

NC, NS = 2, 16
NW = NC * NS
LANES = 16


def _make_lookup(batch, seq_len, latent_dim):
    ch = seq_len // NW
    nsl = latent_dim // LANES
    mesh = plsc.VectorSubcoreMesh(core_axis_name="c", subcore_axis_name="s")

    @functools.partial(
        pl.kernel,
        out_type=jax.ShapeDtypeStruct((batch * seq_len, latent_dim), jnp.float32),
        mesh=mesh,
        scratch_types=[
            pltpu.VMEM((batch, ch), jnp.int32),
            pltpu.VMEM((ch, latent_dim), jnp.float32),
            pltpu.VMEM((ch, latent_dim), jnp.float32),
            pltpu.SemaphoreType.DMA,
        ],
    )
    def body(tok_hbm, emb_hbm, pos_hbm, out_hbm, idx_v, pbuf, buf, sem):
        wid = lax.axis_index("s") * NC + lax.axis_index("c")
        s0 = wid * ch
        pltpu.sync_copy(pos_hbm.at[pl.ds(s0, ch)], pbuf)
        for b in range(batch):
            pltpu.sync_copy(tok_hbm.at[b * NW + wid], idx_v.at[b])
        for b in range(batch):
            pltpu.async_copy(emb_hbm.at[idx_v.at[b]], buf, sem).wait()

            def row(r, _):
                for j in range(nsl):
                    sl = pl.ds(j * LANES, LANES)
                    buf[r, sl] = buf[r, sl] + pbuf[r, sl]
                return 0

            lax.fori_loop(0, ch, row, 0)
            pltpu.sync_copy(buf, out_hbm.at[pl.ds(b * seq_len + s0, ch)])

    return body


def kernel(tokens, embed_table, pos_table):
    b, s = tokens.shape
    v, d = embed_table.shape
    ch = s // NW
    tok = tokens.reshape(b * NW, ch).astype(jnp.int32)
    out = _make_lookup(b, s, d)(tok, embed_table, pos_table)
    return out.reshape(b, s, d)
